# baseline (device time: 8661 ns/iter reference)
import jax
import jax.numpy as jnp
from jax import lax
from jax.experimental import pallas as pl
from jax.experimental.pallas import tpu as pltpu

K = 8
NEG_INF = float("-inf")


def _oddeven_merge(lo, n, r):
    step = r * 2
    if step < n:
        yield from _oddeven_merge(lo, n, step)
        yield from _oddeven_merge(lo + r, n, step)
        for i in range(lo + r, lo + n - r, step):
            yield (i, i + r)
    else:
        yield (lo, lo + r)


def _oddeven_merge_sort(lo, hi):
    if (hi - lo) >= 1:
        mid = lo + ((hi - lo) // 2)
        yield from _oddeven_merge_sort(lo, mid)
        yield from _oddeven_merge_sort(mid + 1, hi)
        yield from _oddeven_merge(lo, hi - lo + 1, 1)


def _merge_sorted8_desc(a, b):
    top = jnp.concatenate(
        [jnp.maximum(a[:, i : i + 1], b[:, K - 1 - i : K - i]) for i in range(K)],
        axis=1,
    )
    hi = jnp.maximum(top[:, :4], top[:, 4:])
    lo = jnp.minimum(top[:, :4], top[:, 4:])
    h1 = jnp.maximum(hi[:, :2], hi[:, 2:])
    h2 = jnp.minimum(hi[:, :2], hi[:, 2:])
    l1 = jnp.maximum(lo[:, :2], lo[:, 2:])
    l2 = jnp.minimum(lo[:, :2], lo[:, 2:])
    quads = [h1, h2, l1, l2]
    cols = []
    for q in quads:
        cols.append(jnp.maximum(q[:, :1], q[:, 1:]))
        cols.append(jnp.minimum(q[:, :1], q[:, 1:]))
    return jnp.concatenate(cols, axis=1)


def _local_top8(x):
    n_chunks = K
    chunk = x.shape[1] // n_chunks
    v = [x[:, i * chunk : (i + 1) * chunk] for i in range(n_chunks)]

    for i, j in _oddeven_merge_sort(0, n_chunks - 1):
        hi = jnp.maximum(v[i], v[j])
        lo = jnp.minimum(v[i], v[j])
        v[i], v[j] = hi, lo

    cols = []
    for k in range(K):
        mk = jnp.max(v[0], axis=1, keepdims=True)
        cols.append(mk)
        depth = K - k
        if depth > 1:
            mask = v[0] == mk
            for j in range(depth - 1):
                v[j] = jnp.where(mask, v[j + 1], v[j])
            v[depth - 1] = jnp.where(mask, NEG_INF, v[depth - 1])
    return jnp.concatenate(cols, axis=1)


def kernel(x):
    m, n = x.shape

    def body(x_ref, out_ref, mine_ref, theirs_ref, send_sem, recv_sem):
        my_x = lax.axis_index("x")
        my_y = lax.axis_index("y")
        my_z = lax.axis_index("z")
        partner = (1 - my_x, my_y, my_z)

        barrier_sem = pltpu.get_barrier_semaphore()
        pl.semaphore_signal(
            barrier_sem, inc=1,
            device_id=partner, device_id_type=pl.DeviceIdType.MESH,
        )

        mine_ref[:, :] = _local_top8(x_ref[:, :])

        pl.semaphore_wait(barrier_sem, 1)

        rdma = pltpu.make_async_remote_copy(
            src_ref=mine_ref,
            dst_ref=theirs_ref,
            send_sem=send_sem,
            recv_sem=recv_sem,
            device_id=partner,
            device_id_type=pl.DeviceIdType.MESH,
        )
        rdma.start()
        rdma.wait()

        out_ref[:, :] = _merge_sorted8_desc(mine_ref[:, :], theirs_ref[:, :])

    return pl.pallas_call(
        body,
        out_shape=jax.ShapeDtypeStruct((m, K), jnp.float32),
        in_specs=[pl.BlockSpec(memory_space=pltpu.VMEM)],
        out_specs=pl.BlockSpec(memory_space=pltpu.VMEM),
        scratch_shapes=[
            pltpu.VMEM((m, K), jnp.float32),
            pltpu.VMEM((m, K), jnp.float32),
            pltpu.SemaphoreType.DMA,
            pltpu.SemaphoreType.DMA,
        ],
        compiler_params=pltpu.CompilerParams(collective_id=0),
    )(x)


# device time: 8656 ns/iter; 1.0006x vs baseline; 1.0006x over previous
import jax
import jax.numpy as jnp
from jax import lax
from jax.experimental import pallas as pl
from jax.experimental.pallas import tpu as pltpu

K = 8
NEG_INF = float("-inf")


def _oddeven_merge(lo, n, r):
    step = r * 2
    if step < n:
        yield from _oddeven_merge(lo, n, step)
        yield from _oddeven_merge(lo + r, n, step)
        for i in range(lo + r, lo + n - r, step):
            yield (i, i + r)
    else:
        yield (lo, lo + r)


def _oddeven_merge_sort(lo, hi):
    if (hi - lo) >= 1:
        mid = lo + ((hi - lo) // 2)
        yield from _oddeven_merge_sort(lo, mid)
        yield from _oddeven_merge_sort(mid + 1, hi)
        yield from _oddeven_merge(lo, hi - lo + 1, 1)


def _merge_sorted8_desc(a, b):
    top = jnp.concatenate(
        [jnp.maximum(a[:, i : i + 1], b[:, K - 1 - i : K - i]) for i in range(K)],
        axis=1,
    )
    hi = jnp.maximum(top[:, :4], top[:, 4:])
    lo = jnp.minimum(top[:, :4], top[:, 4:])
    h1 = jnp.maximum(hi[:, :2], hi[:, 2:])
    h2 = jnp.minimum(hi[:, :2], hi[:, 2:])
    l1 = jnp.maximum(lo[:, :2], lo[:, 2:])
    l2 = jnp.minimum(lo[:, :2], lo[:, 2:])
    quads = [h1, h2, l1, l2]
    cols = []
    for q in quads:
        cols.append(jnp.maximum(q[:, :1], q[:, 1:]))
        cols.append(jnp.minimum(q[:, :1], q[:, 1:]))
    return jnp.concatenate(cols, axis=1)


def _local_top8(x):
    n_chunks = K
    chunk = x.shape[1] // n_chunks
    v = [x[:, i * chunk : (i + 1) * chunk] for i in range(n_chunks)]

    for i, j in _oddeven_merge_sort(0, n_chunks - 1):
        hi = jnp.maximum(v[i], v[j])
        lo = jnp.minimum(v[i], v[j])
        v[i], v[j] = hi, lo

    cols = []
    for k in range(K):
        mk = jnp.max(v[0], axis=1, keepdims=True)
        cols.append(mk)
        depth = K - k
        if depth > 1:
            mask = v[0] == mk
            for j in range(depth - 1):
                v[j] = jnp.where(mask, v[j + 1], v[j])
            v[depth - 1] = jnp.where(mask, NEG_INF, v[depth - 1])
    return jnp.concatenate(cols, axis=1)


def kernel(x):
    m, n = x.shape

    def body(x_ref, out_ref, mine_ref, theirs_ref, send_sem, recv_sem):
        my_x = lax.axis_index("x")
        my_y = lax.axis_index("y")
        my_z = lax.axis_index("z")
        partner = (1 - my_x, my_y, my_z)

        barrier_sem = pltpu.get_barrier_semaphore()
        pl.semaphore_signal(
            barrier_sem, inc=1,
            device_id=partner, device_id_type=pl.DeviceIdType.MESH,
        )

        mine_ref[:, :] = _local_top8(x_ref[:, :])

        pl.semaphore_wait(barrier_sem, 1)

        rdma = pltpu.make_async_remote_copy(
            src_ref=mine_ref,
            dst_ref=theirs_ref,
            send_sem=send_sem,
            recv_sem=recv_sem,
            device_id=partner,
            device_id_type=pl.DeviceIdType.MESH,
        )
        rdma.start()
        rdma.wait_recv()

        out_ref[:, :] = _merge_sorted8_desc(mine_ref[:, :], theirs_ref[:, :])

        rdma.wait_send()

    return pl.pallas_call(
        body,
        out_shape=jax.ShapeDtypeStruct((m, K), jnp.float32),
        in_specs=[pl.BlockSpec(memory_space=pltpu.VMEM)],
        out_specs=pl.BlockSpec(memory_space=pltpu.VMEM),
        scratch_shapes=[
            pltpu.VMEM((m, K), jnp.float32),
            pltpu.VMEM((m, K), jnp.float32),
            pltpu.SemaphoreType.DMA,
            pltpu.SemaphoreType.DMA,
        ],
        compiler_params=pltpu.CompilerParams(collective_id=0),
    )(x)


# device time: 8451 ns/iter; 1.0248x vs baseline; 1.0243x over previous
import jax
import jax.numpy as jnp
from jax import lax
from jax.experimental import pallas as pl
from jax.experimental.pallas import tpu as pltpu

K = 8
NEG_INF = float("-inf")


def _oddeven_merge(lo, n, r):
    step = r * 2
    if step < n:
        yield from _oddeven_merge(lo, n, step)
        yield from _oddeven_merge(lo + r, n, step)
        for i in range(lo + r, lo + n - r, step):
            yield (i, i + r)
    else:
        yield (lo, lo + r)


def _oddeven_merge_sort(lo, hi):
    if (hi - lo) >= 1:
        mid = lo + ((hi - lo) // 2)
        yield from _oddeven_merge_sort(lo, mid)
        yield from _oddeven_merge_sort(mid + 1, hi)
        yield from _oddeven_merge(lo, hi - lo + 1, 1)


def _merge_sorted8_desc(a, b):
    vals = jnp.concatenate([a, b], axis=1)
    cols = []
    for _ in range(K):
        mk = jnp.max(vals, axis=1, keepdims=True)
        cols.append(mk)
        vals = jnp.where(vals == mk, NEG_INF, vals)
    return jnp.concatenate(cols, axis=1)


def _local_top8(x):
    n_chunks = K
    chunk = x.shape[1] // n_chunks
    v = [x[:, i * chunk : (i + 1) * chunk] for i in range(n_chunks)]

    for i, j in _oddeven_merge_sort(0, n_chunks - 1):
        hi = jnp.maximum(v[i], v[j])
        lo = jnp.minimum(v[i], v[j])
        v[i], v[j] = hi, lo

    cols = []
    for k in range(K):
        mk = jnp.max(v[0], axis=1, keepdims=True)
        cols.append(mk)
        depth = K - k
        if depth > 1:
            mask = v[0] == mk
            for j in range(depth - 1):
                v[j] = jnp.where(mask, v[j + 1], v[j])
            v[depth - 1] = jnp.where(mask, NEG_INF, v[depth - 1])
    return jnp.concatenate(cols, axis=1)


def kernel(x):
    m, n = x.shape

    def body(x_ref, out_ref, mine_ref, theirs_ref, send_sem, recv_sem):
        my_x = lax.axis_index("x")
        my_y = lax.axis_index("y")
        my_z = lax.axis_index("z")
        partner = (1 - my_x, my_y, my_z)

        barrier_sem = pltpu.get_barrier_semaphore()
        pl.semaphore_signal(
            barrier_sem, inc=1,
            device_id=partner, device_id_type=pl.DeviceIdType.MESH,
        )

        mine_ref[:, :] = _local_top8(x_ref[:, :])

        pl.semaphore_wait(barrier_sem, 1)

        rdma = pltpu.make_async_remote_copy(
            src_ref=mine_ref,
            dst_ref=theirs_ref,
            send_sem=send_sem,
            recv_sem=recv_sem,
            device_id=partner,
            device_id_type=pl.DeviceIdType.MESH,
        )
        rdma.start()
        rdma.wait_recv()

        out_ref[:, :] = _merge_sorted8_desc(mine_ref[:, :], theirs_ref[:, :])

        rdma.wait_send()

    return pl.pallas_call(
        body,
        out_shape=jax.ShapeDtypeStruct((m, K), jnp.float32),
        in_specs=[pl.BlockSpec(memory_space=pltpu.VMEM)],
        out_specs=pl.BlockSpec(memory_space=pltpu.VMEM),
        scratch_shapes=[
            pltpu.VMEM((m, K), jnp.float32),
            pltpu.VMEM((m, K), jnp.float32),
            pltpu.SemaphoreType.DMA,
            pltpu.SemaphoreType.DMA,
        ],
        compiler_params=pltpu.CompilerParams(collective_id=0),
    )(x)
